# baseline (speedup 1.0000x reference)
import functools

import jax
import jax.numpy as jnp
from jax import lax
from jax.experimental import pallas as pl
from jax.experimental.pallas import tpu as pltpu
from jax.experimental.pallas import tpu_sc as plsc

_NC = 2
_NS = 16
_L = 16


def _matmul_call(h, wbig, bias):
    n, d_in = h.shape
    d_out = wbig.shape[1]
    d = d_out // 4
    bn = 1000

    def body(h_ref, w_ref, b_ref, base_ref, z_ref):
        y = jnp.dot(h_ref[...], w_ref[...],
                    preferred_element_type=jnp.float32) + b_ref[...]
        base_ref[...] = y[:, :d]
        z_ref[...] = y[:, d:]

    return pl.pallas_call(
        body,
        grid=(n // bn,),
        in_specs=[
            pl.BlockSpec((bn, d_in), lambda i: (i, 0)),
            pl.BlockSpec((d_in, d_out), lambda i: (0, 0)),
            pl.BlockSpec((1, d_out), lambda i: (0, 0)),
        ],
        out_specs=[
            pl.BlockSpec((bn, d), lambda i: (i, 0)),
            pl.BlockSpec((bn, 3 * d), lambda i: (i, 0)),
        ],
        out_shape=[
            jax.ShapeDtypeStruct((n, d), jnp.float32),
            jax.ShapeDtypeStruct((n, 3 * d), jnp.float32),
        ],
    )(h, wbig, bias)


def _sc_scatter_call(z3, edata, distf, n_nodes, e):
    d = z3.shape[1]
    nw = _NC * _NS
    epw = e // nw
    c = 128
    n_chunks = epw // c
    assert n_chunks % 2 == 1 and n_chunks >= 3
    n_pairs = (n_chunks - 1) // 2
    rps = (n_nodes // _NS) & ~7
    oc = 16
    mesh = plsc.VectorSubcoreMesh(core_axis_name="c", subcore_axis_name="s")

    @functools.partial(
        pl.kernel,
        mesh=mesh,
        out_type=jax.ShapeDtypeStruct((_NC * n_nodes, d), jnp.float32),
        scratch_types=[
            pltpu.VMEM((3 * c,), jnp.int32),
            pltpu.VMEM((3 * c,), jnp.int32),
            pltpu.VMEM((c,), jnp.float32),
            pltpu.VMEM((c,), jnp.float32),
            pltpu.VMEM((c,), jnp.int32),
            pltpu.VMEM((c,), jnp.int32),
            pltpu.VMEM((c,), jnp.int32),
            pltpu.VMEM((c,), jnp.int32),
            pltpu.VMEM((c,), jnp.float32),
            pltpu.VMEM((c,), jnp.float32),
            pltpu.VMEM((c, d), jnp.float32),
            pltpu.VMEM((c, d), jnp.float32),
            pltpu.VMEM_SHARED((n_nodes, d), jnp.float32),
            pltpu.SemaphoreType.DMA,
            pltpu.SemaphoreType.DMA,
            pltpu.SemaphoreType.DMA,
            pltpu.SemaphoreType.DMA,
        ],
    )
    def k(z_hbm, edata_hbm, dist_hbm, out_hbm,
          ebuf0, ebuf1, dbuf0, dbuf1, idx0, idx1, sct0, sct1,
          scl0, scl1, rows0, rows1, acc_sh, gsem0, gsem1, ssem0, ssem1):
        cid = lax.axis_index("c")
        sid = lax.axis_index("s")
        wid = cid * _NS + sid

        bufs = ((idx0, sct0, scl0, rows0, gsem0, ssem0, ebuf0, dbuf0),
                (idx1, sct1, scl1, rows1, gsem1, ssem1, ebuf1, dbuf1))

        def zero_rows(j, carry):
            for kk in range(d // _L):
                rows0[j, pl.ds(kk * _L, _L)] = jnp.zeros((_L,), jnp.float32)
            return carry

        lax.fori_loop(0, oc, zero_rows, 0)
        r0 = sid * rps
        nrows = jnp.where(sid == _NS - 1, n_nodes - (_NS - 1) * rps, rps)

        def zcopy(i, carry):
            pltpu.sync_copy(rows0.at[pl.ds(0, oc)],
                            acc_sh.at[pl.ds(r0 + i * oc, oc)])
            return carry

        lax.fori_loop(0, nrows // oc, zcopy, 0)
        plsc.subcore_barrier()

        def build(t, bi):
            idx_v, sct_v, scl_v = bufs[bi][0], bufs[bi][1], bufs[bi][2]
            ebuf, dbuf = bufs[bi][6], bufs[bi][7]
            gc = wid * n_chunks + t
            pltpu.sync_copy(edata_hbm.at[pl.ds(gc * (3 * c), 3 * c)], ebuf)
            pltpu.sync_copy(dist_hbm.at[pl.ds(gc * c, c)], dbuf)
            for i in range(c // _L):
                s16 = ebuf[pl.ds(i * _L, _L)]
                t16 = ebuf[pl.ds(c + i * _L, _L)]
                p16 = ebuf[pl.ds(2 * c + i * _L, _L)]
                d16 = dbuf[pl.ds(i * _L, _L)]
                idx_v[pl.ds(i * _L, _L)] = s16 * 3 + jnp.minimum(p16, 2)
                scl_v[pl.ds(i * _L, _L)] = jnp.where(p16 == 3, 0.0, d16)
                sct_v[pl.ds(i * _L, _L)] = t16

        def start_gather(bi):
            pltpu.async_copy(z_hbm.at[bufs[bi][0]], bufs[bi][3], bufs[bi][4])

        def wait_gather(bi):
            pltpu.make_async_copy(
                z_hbm.at[bufs[bi][0]], bufs[bi][3], bufs[bi][4]).wait()

        def start_scatter(bi):
            pltpu.async_copy(bufs[bi][3], acc_sh.at[bufs[bi][1]],
                             bufs[bi][5], add=True)

        def wait_scatter(bi):
            pltpu.make_async_copy(
                bufs[bi][3], acc_sh.at[bufs[bi][1]], bufs[bi][5]).wait()

        def scale(bi):
            scl_v, rows_v = bufs[bi][2], bufs[bi][3]

            def scale_grp(g, cry):
                s16 = scl_v[pl.ds(g * _L, _L)]
                for jj in range(_L):
                    s = s16[jj]
                    j = g * _L + jj
                    for kk in range(d // _L):
                        rows_v[j, pl.ds(kk * _L, _L)] = (
                            rows_v[j, pl.ds(kk * _L, _L)] * s)
                return cry

            lax.fori_loop(0, c // _L, scale_grp, 0)

        build(0, 0)
        start_gather(0)

        def pair_body(p, carry):
            wait_gather(0)
            scale(0)
            start_scatter(0)

            @pl.when(p > 0)
            def _():
                wait_scatter(1)
            build(2 * p + 1, 1)
            start_gather(1)

            wait_gather(1)
            scale(1)
            start_scatter(1)

            wait_scatter(0)
            build(2 * p + 2, 0)
            start_gather(0)
            return carry

        lax.fori_loop(0, n_pairs, pair_body, 0)
        wait_gather(0)
        scale(0)
        start_scatter(0)
        wait_scatter(1)
        wait_scatter(0)
        plsc.subcore_barrier()

        def out_copy(i, carry):
            pltpu.sync_copy(acc_sh.at[pl.ds(r0 + i * oc, oc)],
                            rows0.at[pl.ds(0, oc)])
            pltpu.sync_copy(
                rows0.at[pl.ds(0, oc)],
                out_hbm.at[pl.ds(cid * n_nodes + r0 + i * oc, oc)])
            return carry

        lax.fori_loop(0, nrows // oc, out_copy, 0)

    return k(z3, edata, distf)


def _epilogue_call(basearr, partials, gamma, beta):
    n, d = basearr.shape
    bn = 1000
    nblk = n // bn

    def body(b_ref, p0_ref, p1_ref, g_ref, be_ref, o_ref):
        y = b_ref[...] + p0_ref[...] + p1_ref[...]
        mu = jnp.mean(y, axis=-1, keepdims=True)
        var = jnp.mean(jnp.square(y - mu), axis=-1, keepdims=True)
        yn = (y - mu) * lax.rsqrt(var + 1e-5) * g_ref[...] + be_ref[...]
        o_ref[...] = jnp.maximum(yn, 0.0)

    return pl.pallas_call(
        body,
        grid=(nblk,),
        in_specs=[
            pl.BlockSpec((bn, d), lambda i: (i, 0)),
            pl.BlockSpec((bn, d), lambda i: (i, 0)),
            pl.BlockSpec((bn, d), lambda i: (i + nblk, 0)),
            pl.BlockSpec((1, d), lambda i: (0, 0)),
            pl.BlockSpec((1, d), lambda i: (0, 0)),
        ],
        out_specs=pl.BlockSpec((bn, d), lambda i: (i, 0)),
        out_shape=jax.ShapeDtypeStruct((n, d), jnp.float32),
    )(basearr, partials, partials, gamma.reshape(1, d), beta.reshape(1, d))


def kernel(h, edge_index, positions, dist, W, b, gamma, beta):
    n, d = h.shape
    wstack = jnp.concatenate(
        [W[:, :d], W[:, d:2 * d] + W[:, 4 * d:], W[:, 2 * d:3 * d],
         W[:, 3 * d:4 * d]], axis=0)
    wbig = wstack.T
    bias = jnp.concatenate(
        [b, jnp.zeros((3 * d,), jnp.float32)]).reshape(1, 4 * d)

    basearr, z = _matmul_call(h, wbig, bias)
    z3 = z.reshape(3 * n, d)

    e = edge_index.shape[1]
    c = 128
    nw = _NC * _NS
    e_pad = -(-e // (nw * c)) * (nw * c)
    pad = e_pad - e
    srcp = jnp.concatenate([edge_index[0], jnp.zeros((pad,), jnp.int32)])
    dstp = jnp.concatenate([edge_index[1], jnp.zeros((pad,), jnp.int32)])
    posp = jnp.concatenate([positions, jnp.full((pad,), 3, jnp.int32)])
    distf = jnp.concatenate([dist.reshape(-1),
                             jnp.zeros((pad,), jnp.float32)])
    edata = jnp.stack(
        [srcp.reshape(-1, c), dstp.reshape(-1, c), posp.reshape(-1, c)],
        axis=1).reshape(-1)

    partials = _sc_scatter_call(z3, edata, distf, n, e_pad)
    return _epilogue_call(basearr, partials, gamma, beta)


# speedup vs baseline: 1.7456x; 1.7456x over previous
import functools

import jax
import jax.numpy as jnp
from jax import lax
from jax.experimental import pallas as pl
from jax.experimental.pallas import tpu as pltpu
from jax.experimental.pallas import tpu_sc as plsc

_NC = 2
_NS = 16
_L = 16


def _matmul_call(h, wbig, bias):
    n, d_in = h.shape
    d_out = wbig.shape[1]
    d = d_out // 4
    bn = 1000

    def body(h_ref, w_ref, b_ref, base_ref, z_ref):
        y = jnp.dot(h_ref[...], w_ref[...],
                    preferred_element_type=jnp.float32) + b_ref[...]
        base_ref[...] = y[:, :d]
        z_ref[...] = y[:, d:]

    return pl.pallas_call(
        body,
        grid=(n // bn,),
        in_specs=[
            pl.BlockSpec((bn, d_in), lambda i: (i, 0)),
            pl.BlockSpec((d_in, d_out), lambda i: (0, 0)),
            pl.BlockSpec((1, d_out), lambda i: (0, 0)),
        ],
        out_specs=[
            pl.BlockSpec((bn, d), lambda i: (i, 0)),
            pl.BlockSpec((bn, 3 * d), lambda i: (i, 0)),
        ],
        out_shape=[
            jax.ShapeDtypeStruct((n, d), jnp.float32),
            jax.ShapeDtypeStruct((n, 3 * d), jnp.float32),
        ],
    )(h, wbig, bias)


def _sc_scatter_call(z3, packed, scale, n_nodes, e):
    d = z3.shape[1]
    nw = _NC * _NS
    epw = e // nw
    c = 80
    n_chunks = epw // c
    assert n_chunks % 2 == 1 and n_chunks >= 3
    n_pairs = (n_chunks - 1) // 2
    rps = (n_nodes // _NS) & ~7
    oc = 16
    mesh = plsc.VectorSubcoreMesh(core_axis_name="c", subcore_axis_name="s")

    @functools.partial(
        pl.kernel,
        mesh=mesh,
        out_type=jax.ShapeDtypeStruct((_NC * n_nodes, d), jnp.float32),
        scratch_types=[
            pltpu.VMEM((epw,), jnp.int32),
            pltpu.VMEM((epw,), jnp.float32),
            pltpu.VMEM((c,), jnp.int32),
            pltpu.VMEM((c,), jnp.int32),
            pltpu.VMEM((c,), jnp.int32),
            pltpu.VMEM((c,), jnp.int32),
            pltpu.VMEM((c, d), jnp.float32),
            pltpu.VMEM((c, d), jnp.float32),
            pltpu.VMEM_SHARED((n_nodes, d), jnp.float32),
            pltpu.SemaphoreType.DMA,
            pltpu.SemaphoreType.DMA,
            pltpu.SemaphoreType.DMA,
            pltpu.SemaphoreType.DMA,
        ],
    )
    def k(z_hbm, pkd_hbm, scl_hbm, out_hbm,
          pkd_a, scl_a, idx0, idx1, sct0, sct1,
          rows0, rows1, acc_sh, gsem0, gsem1, ssem0, ssem1):
        cid = lax.axis_index("c")
        sid = lax.axis_index("s")
        wid = cid * _NS + sid
        base = wid * epw

        bufs = ((idx0, sct0, rows0, gsem0, ssem0),
                (idx1, sct1, rows1, gsem1, ssem1))

        pltpu.sync_copy(pkd_hbm.at[pl.ds(base, epw)], pkd_a)
        pltpu.sync_copy(scl_hbm.at[pl.ds(base, epw)], scl_a)

        def zero_rows(j, carry):
            for kk in range(d // _L):
                rows0[j, pl.ds(kk * _L, _L)] = jnp.zeros((_L,), jnp.float32)
            return carry

        lax.fori_loop(0, oc, zero_rows, 0)
        r0 = sid * rps
        nrows = jnp.where(sid == _NS - 1, n_nodes - (_NS - 1) * rps, rps)

        def zcopy(i, carry):
            pltpu.sync_copy(rows0.at[pl.ds(0, oc)],
                            acc_sh.at[pl.ds(r0 + i * oc, oc)])
            return carry

        lax.fori_loop(0, nrows // oc, zcopy, 0)
        plsc.subcore_barrier()

        def build(t, bi):
            idx_v, sct_v = bufs[bi][0], bufs[bi][1]
            off = t * c
            for i in range(c // _L):
                w16 = pkd_a[pl.ds(off + i * _L, _L)]
                idx_v[pl.ds(i * _L, _L)] = w16 & 0x7FFF
                sct_v[pl.ds(i * _L, _L)] = lax.shift_right_logical(w16, 15)

        def start_gather(bi):
            pltpu.async_copy(z_hbm.at[bufs[bi][0]], bufs[bi][2], bufs[bi][3])

        def wait_gather(bi):
            pltpu.make_async_copy(
                z_hbm.at[bufs[bi][0]], bufs[bi][2], bufs[bi][3]).wait()

        def start_scatter(bi):
            pltpu.async_copy(bufs[bi][2], acc_sh.at[bufs[bi][1]],
                             bufs[bi][4], add=True)

        def wait_scatter(bi):
            pltpu.make_async_copy(
                bufs[bi][2], acc_sh.at[bufs[bi][1]], bufs[bi][4]).wait()

        def scale(t, bi):
            rows_v = bufs[bi][2]
            off = t * c

            def scale_grp(g, cry):
                s16 = scl_a[pl.ds(off + g * _L, _L)]
                for jj in range(_L):
                    s = s16[jj]
                    j = g * _L + jj
                    for kk in range(d // _L):
                        rows_v[j, pl.ds(kk * _L, _L)] = (
                            rows_v[j, pl.ds(kk * _L, _L)] * s)
                return cry

            lax.fori_loop(0, c // _L, scale_grp, 0)

        build(0, 0)
        start_gather(0)

        def pair_body(p, carry):
            wait_gather(0)
            scale(2 * p, 0)
            start_scatter(0)

            @pl.when(p > 0)
            def _():
                wait_scatter(1)
            build(2 * p + 1, 1)
            start_gather(1)

            wait_gather(1)
            scale(2 * p + 1, 1)
            start_scatter(1)

            wait_scatter(0)
            build(2 * p + 2, 0)
            start_gather(0)
            return carry

        lax.fori_loop(0, n_pairs, pair_body, 0)
        wait_gather(0)
        scale(n_chunks - 1, 0)
        start_scatter(0)
        wait_scatter(1)
        wait_scatter(0)
        plsc.subcore_barrier()

        def out_copy(i, carry):
            pltpu.sync_copy(acc_sh.at[pl.ds(r0 + i * oc, oc)],
                            rows0.at[pl.ds(0, oc)])
            pltpu.sync_copy(
                rows0.at[pl.ds(0, oc)],
                out_hbm.at[pl.ds(cid * n_nodes + r0 + i * oc, oc)])
            return carry

        lax.fori_loop(0, nrows // oc, out_copy, 0)

    return k(z3, packed, scale)


def _epilogue_call(basearr, partials, gamma, beta):
    n, d = basearr.shape
    bn = 1000
    nblk = n // bn

    def body(b_ref, p0_ref, p1_ref, g_ref, be_ref, o_ref):
        y = b_ref[...] + p0_ref[...] + p1_ref[...]
        mu = jnp.mean(y, axis=-1, keepdims=True)
        var = jnp.mean(jnp.square(y - mu), axis=-1, keepdims=True)
        yn = (y - mu) * lax.rsqrt(var + 1e-5) * g_ref[...] + be_ref[...]
        o_ref[...] = jnp.maximum(yn, 0.0)

    return pl.pallas_call(
        body,
        grid=(nblk,),
        in_specs=[
            pl.BlockSpec((bn, d), lambda i: (i, 0)),
            pl.BlockSpec((bn, d), lambda i: (i, 0)),
            pl.BlockSpec((bn, d), lambda i: (i + nblk, 0)),
            pl.BlockSpec((1, d), lambda i: (0, 0)),
            pl.BlockSpec((1, d), lambda i: (0, 0)),
        ],
        out_specs=pl.BlockSpec((bn, d), lambda i: (i, 0)),
        out_shape=jax.ShapeDtypeStruct((n, d), jnp.float32),
    )(basearr, partials, partials, gamma.reshape(1, d), beta.reshape(1, d))


def kernel(h, edge_index, positions, dist, W, b, gamma, beta):
    n, d = h.shape
    wstack = jnp.concatenate(
        [W[:, :d], W[:, d:2 * d] + W[:, 4 * d:], W[:, 2 * d:3 * d],
         W[:, 3 * d:4 * d]], axis=0)
    wbig = wstack.T
    bias = jnp.concatenate(
        [b, jnp.zeros((3 * d,), jnp.float32)]).reshape(1, 4 * d)

    basearr, z = _matmul_call(h, wbig, bias)
    z3 = z.reshape(3 * n, d)

    e = edge_index.shape[1]
    nw = _NC * _NS
    e_pad = -(-e // (nw * 80)) * (nw * 80)
    pad = e_pad - e
    idx = 3 * edge_index[0] + jnp.minimum(positions, 2)
    word = idx | (edge_index[1] << 15)
    packed = jnp.concatenate([word, jnp.zeros((pad,), jnp.int32)])
    scl = jnp.where(positions == 3, 0.0, dist.reshape(-1))
    scale = jnp.concatenate([scl, jnp.zeros((pad,), jnp.float32)])

    partials = _sc_scatter_call(z3, packed, scale, n, e_pad)
    return _epilogue_call(basearr, partials, gamma, beta)
